# baseline (speedup 1.0000x reference)
import functools

import numpy as np
import jax
import jax.numpy as jnp
from jax import lax
from jax.experimental import pallas as pl
from jax.experimental.pallas import tpu as pltpu
from jax.experimental.pallas import tpu_sc as plsc

_LOG_EPS = float(np.log(np.float32(1e-9)))


def _gate_logits_kernel(emb_ref, wg_ref, bg_ref, gl_ref):
    x = emb_ref[...]
    gl_ref[...] = jax.lax.dot_general(
        wg_ref[...], x, (((0,), (1,)), ((), ())),
        preferred_element_type=jnp.float32) + bg_ref[...]


def _experts_kernel(emb_ref, w1_ref, b1_ref, w2_ref, b2_ref,
                    projs_ref, *, n_experts: int, proj_dim: int):
    x = emb_ref[...]
    h = jax.lax.dot_general(
        x, w1_ref[...], (((1,), (0,)), ((), ())),
        preferred_element_type=jnp.float32) + b1_ref[...]
    h = jnp.maximum(h, 0.0)
    for e in range(n_experts):
        he = h[:, e * proj_dim:(e + 1) * proj_dim]
        out = jax.lax.dot_general(
            he, w2_ref[e], (((1,), (0,)), ((), ())),
            preferred_element_type=jnp.float32)
        projs_ref[:, e * proj_dim:(e + 1) * proj_dim] = (
            out + b2_ref[:, e * proj_dim:(e + 1) * proj_dim])


def _make_sc_routing(E: int, B: int):
    info = plsc.get_sparse_core_info()
    nc, ns, L = info.num_cores, info.num_subcores, info.num_lanes
    nw = nc * ns
    chunk = B // nw
    mesh = plsc.VectorSubcoreMesh(core_axis_name="c", subcore_axis_name="s")

    @functools.partial(
        pl.kernel, mesh=mesh,
        out_type=jax.ShapeDtypeStruct((E * B,), jnp.float32),
        scratch_types=[pltpu.VMEM((E * chunk,), jnp.float32),
                       pltpu.VMEM((E * chunk,), jnp.float32)],
    )
    def routing(gl_hbm, out_hbm, gl_v, out_v):
        wid = lax.axis_index("s") * nc + lax.axis_index("c")
        base = wid * chunk
        for e in range(E):
            pltpu.sync_copy(gl_hbm.at[pl.ds(e * B + base, chunk)],
                            gl_v.at[pl.ds(e * chunk, chunk)])
        for j in range(chunk // L):
            rows = [gl_v[pl.ds(e * chunk + j * L, L)] for e in range(E)]
            adj = []
            for i in range(E):
                rank = None
                for jj in range(E):
                    if jj == i:
                        continue
                    beats = rows[jj] > rows[i]
                    if jj < i:
                        beats = beats | (rows[jj] == rows[i])
                    b = jnp.where(beats, 1.0, 0.0)
                    rank = b if rank is None else rank + b
                keep = rank < 2.0
                adj.append(rows[i] + jnp.where(keep, 0.0, _LOG_EPS))
            m = adj[0]
            for i in range(1, E):
                m = jnp.maximum(m, adj[i])
            ex = [jnp.exp(a - m) for a in adj]
            s = ex[0]
            for i in range(1, E):
                s = s + ex[i]
            for i in range(E):
                out_v[pl.ds(i * chunk + j * L, L)] = ex[i] / s
        for e in range(E):
            pltpu.sync_copy(out_v.at[pl.ds(e * chunk, chunk)],
                            out_hbm.at[pl.ds(e * B + base, chunk)])

    return routing


def kernel(emb, Wg, bg, W1, b1, W2, b2, top_k):
    del top_k
    B, in_dim = emb.shape
    E = Wg.shape[1]
    P = W2.shape[-1]

    BT = min(1024, B)
    assert B % BT == 0

    W1f = W1.transpose(1, 0, 2).reshape(in_dim, E * P)
    b1f = b1.reshape(1, E * P)
    b2f = b2.reshape(1, E * P)
    bg2 = bg.reshape(E, 1)

    grid = (B // BT,)

    gl = pl.pallas_call(
        _gate_logits_kernel,
        grid=grid,
        in_specs=[
            pl.BlockSpec((BT, in_dim), lambda i: (i, 0)),
            pl.BlockSpec((in_dim, E), lambda i: (0, 0)),
            pl.BlockSpec((E, 1), lambda i: (0, 0)),
        ],
        out_specs=pl.BlockSpec((E, BT), lambda i: (0, i)),
        out_shape=jax.ShapeDtypeStruct((E, B), jnp.float32),
        compiler_params=pltpu.CompilerParams(
            dimension_semantics=("arbitrary",)),
    )(emb, Wg, bg2)

    gate_wt = _make_sc_routing(E, B)(gl.reshape(E * B))

    projs2d = pl.pallas_call(
        functools.partial(_experts_kernel, n_experts=E, proj_dim=P),
        grid=grid,
        in_specs=[
            pl.BlockSpec((BT, in_dim), lambda i: (i, 0)),
            pl.BlockSpec((in_dim, E * P), lambda i: (0, 0)),
            pl.BlockSpec((1, E * P), lambda i: (0, 0)),
            pl.BlockSpec((E, P, P), lambda i: (0, 0, 0)),
            pl.BlockSpec((1, E * P), lambda i: (0, 0)),
        ],
        out_specs=pl.BlockSpec((BT, E * P), lambda i: (i, 0)),
        out_shape=jax.ShapeDtypeStruct((B, E * P), jnp.float32),
        compiler_params=pltpu.CompilerParams(
            dimension_semantics=("arbitrary",)),
    )(emb, W1f, b1f, W2, b2f)

    return projs2d.reshape(B, E, P), gate_wt.reshape(E, B).T


# speedup vs baseline: 1.3156x; 1.3156x over previous
import functools

import jax
import jax.numpy as jnp
from jax.experimental import pallas as pl
from jax.experimental.pallas import tpu as pltpu


def _fused_kernel(emb_ref, wg_ref, bg_ref, w1_ref, b1_ref, w2_ref, b2_ref,
                  projs_ref, gate_ref, *, n_experts: int, proj_dim: int):
    x = emb_ref[...]

    gt = jax.lax.dot_general(
        wg_ref[...], x, (((0,), (1,)), ((), ())),
        preferred_element_type=jnp.float32) + bg_ref[...]
    rows = [gt[i:i + 1, :] for i in range(n_experts)]
    adj = []
    for i in range(n_experts):
        rank = None
        for j in range(n_experts):
            if j == i:
                continue
            beats = rows[j] > rows[i]
            if j < i:
                beats = beats | (rows[j] == rows[i])
            b = beats.astype(jnp.float32)
            rank = b if rank is None else rank + b
        mask = (rank < 2.0).astype(jnp.float32)
        adj.append(rows[i] + jnp.log(mask + 1e-9))
    m = adj[0]
    for i in range(1, n_experts):
        m = jnp.maximum(m, adj[i])
    ex = [jnp.exp(a - m) for a in adj]
    s = ex[0]
    for i in range(1, n_experts):
        s = s + ex[i]
    gate_ref[...] = jnp.concatenate([e / s for e in ex], axis=0)

    h = jax.lax.dot_general(
        x, w1_ref[...], (((1,), (0,)), ((), ())),
        preferred_element_type=jnp.float32) + b1_ref[...]
    h = jnp.maximum(h, 0.0)
    for e in range(n_experts):
        he = h[:, e * proj_dim:(e + 1) * proj_dim]
        out = jax.lax.dot_general(
            he, w2_ref[e], (((1,), (0,)), ((), ())),
            preferred_element_type=jnp.float32)
        projs_ref[:, e * proj_dim:(e + 1) * proj_dim] = (
            out + b2_ref[:, e * proj_dim:(e + 1) * proj_dim])


def kernel(emb, Wg, bg, W1, b1, W2, b2, top_k):
    del top_k
    B, in_dim = emb.shape
    E = Wg.shape[1]
    P = W2.shape[-1]

    BT = min(1024, B)
    assert B % BT == 0

    W1f = W1.transpose(1, 0, 2).reshape(in_dim, E * P)
    b1f = b1.reshape(1, E * P)
    b2f = b2.reshape(1, E * P)
    bg2 = bg.reshape(E, 1)

    grid = (B // BT,)
    projs2d, gate_w = pl.pallas_call(
        functools.partial(_fused_kernel, n_experts=E, proj_dim=P),
        grid=grid,
        in_specs=[
            pl.BlockSpec((BT, in_dim), lambda i: (i, 0)),
            pl.BlockSpec((in_dim, E), lambda i: (0, 0)),
            pl.BlockSpec((E, 1), lambda i: (0, 0)),
            pl.BlockSpec((in_dim, E * P), lambda i: (0, 0)),
            pl.BlockSpec((1, E * P), lambda i: (0, 0)),
            pl.BlockSpec((E, P, P), lambda i: (0, 0, 0)),
            pl.BlockSpec((1, E * P), lambda i: (0, 0)),
        ],
        out_specs=[
            pl.BlockSpec((BT, E * P), lambda i: (i, 0)),
            pl.BlockSpec((E, BT), lambda i: (0, i)),
        ],
        out_shape=[
            jax.ShapeDtypeStruct((B, E * P), jnp.float32),
            jax.ShapeDtypeStruct((E, B), jnp.float32),
        ],
        compiler_params=pltpu.CompilerParams(
            dimension_semantics=("parallel",)),
    )(emb, Wg, bg2, W1f, b1f, W2, b2f)

    return projs2d.reshape(B, E, P), gate_w.T
